# Initial kernel scaffold; baseline (speedup 1.0000x reference)
import jax
import jax.numpy as jnp
from jax.experimental import pallas as pl


def kernel(x, adj, W1, b1, W2, b2):
    raise NotImplementedError("write your pallas kernel here")


# speedup vs baseline: 4.8217x; 4.8217x over previous
import jax
import jax.numpy as jnp
from jax import lax
from jax.experimental import pallas as pl
from jax.experimental.pallas import tpu as pltpu
from jax.experimental.pallas import tpu_sc as plsc

_NC = 2
_NS = 16
_NW = _NC * _NS
_K = 128
_L = 16


def _make_segsum(n_rows, n_pad, f, n_chunks, with_deg):
    stripe = n_pad // _NS
    mesh = plsc.VectorSubcoreMesh(core_axis_name="core", subcore_axis_name="subcore")
    out_type = [jax.ShapeDtypeStruct((_NC, n_pad, f), jnp.float32)]
    scratch = [
        pltpu.VMEM((n_chunks, _K), jnp.int32),
        pltpu.VMEM((n_chunks, _K), jnp.int32),
        pltpu.VMEM((_K, f), jnp.float32),
        pltpu.VMEM((64, f), jnp.float32),
        pltpu.VMEM_SHARED((n_pad, f), jnp.float32),
    ]
    if with_deg:
        out_type.append(jax.ShapeDtypeStruct((_NC, n_pad), jnp.float32))
        scratch += [
            pltpu.VMEM((_K,), jnp.float32),
            pltpu.VMEM((stripe,), jnp.float32),
            pltpu.VMEM_SHARED((n_pad,), jnp.float32),
        ]

    def body(vals, srcp, dstp, out, *rest):
        if with_deg:
            deg_out, idxs, idxd, rows, zbuf, acc, ones, zdeg, accd = rest
        else:
            idxs, idxd, rows, zbuf, acc = rest
        c = lax.axis_index("core")
        s = lax.axis_index("subcore")
        wid = c * _NS + s

        pltpu.sync_copy(srcp.at[wid], idxs)
        pltpu.sync_copy(dstp.at[wid], idxd)

        zvec = jnp.zeros((_L,), jnp.float32)

        @pl.loop(0, 64)
        def _(i):
            @pl.loop(0, f, step=_L)
            def _(j):
                zbuf[i, pl.ds(j, _L)] = zvec

        @pl.loop(0, stripe, step=64)
        def _(r):
            pltpu.sync_copy(zbuf, acc.at[pl.ds(s * stripe + r, 64)])

        if with_deg:
            ovec = jnp.ones((_L,), jnp.float32)

            @pl.loop(0, _K, step=_L)
            def _(j):
                ones[pl.ds(j, _L)] = ovec

            @pl.loop(0, stripe, step=_L)
            def _(j):
                zdeg[pl.ds(j, _L)] = zvec

            pltpu.sync_copy(zdeg, accd.at[pl.ds(s * stripe, stripe)])

        plsc.subcore_barrier()

        @pl.loop(0, n_chunks)
        def _(j):
            pltpu.sync_copy(vals.at[idxs.at[j]], rows)
            pltpu.sync_copy(rows, acc.at[idxd.at[j]], add=True)
            if with_deg:
                pltpu.sync_copy(ones, accd.at[idxd.at[j]], add=True)

        plsc.subcore_barrier()

        pltpu.sync_copy(acc.at[pl.ds(s * stripe, stripe)],
                        out.at[c, pl.ds(s * stripe, stripe)])
        if with_deg:
            pltpu.sync_copy(accd.at[pl.ds(s * stripe, stripe)],
                            deg_out.at[c, pl.ds(s * stripe, stripe)])

    return pl.kernel(body, out_type=tuple(out_type), mesh=mesh,
                     scratch_types=scratch)


def _tc_layer1(s1, deg3, W1, b1, blk):
    n_pad, f_in = s1.shape[1], s1.shape[2]
    h_dim = W1.shape[1]

    def body(p_ref, d_ref, w1_ref, b1_ref, o_ref):
        ssum = p_ref[0] + p_ref[1]
        dsum = d_ref[0] + d_ref[1]
        norm = 1.0 / jnp.maximum(dsum, 1.0)
        agg = jnp.dot(ssum, w1_ref[...], preferred_element_type=jnp.float32)
        agg = (agg + dsum * b1_ref[...]) * norm
        o_ref[...] = jnp.maximum(agg, 0.0)

    return pl.pallas_call(
        body,
        grid=(n_pad // blk,),
        in_specs=[
            pl.BlockSpec((_NC, blk, f_in), lambda i: (0, i, 0)),
            pl.BlockSpec((_NC, blk, 1), lambda i: (0, i, 0)),
            pl.BlockSpec((f_in, h_dim), lambda i: (0, 0)),
            pl.BlockSpec((1, h_dim), lambda i: (0, 0)),
        ],
        out_specs=pl.BlockSpec((blk, h_dim), lambda i: (i, 0)),
        out_shape=jax.ShapeDtypeStruct((n_pad, h_dim), jnp.float32),
    )(s1, deg3, W1, b1.reshape(1, h_dim))


def _tc_layer2(s2, deg3, W2, b2, blk):
    n_pad, h_dim = s2.shape[1], s2.shape[2]
    c_dim = W2.shape[1]

    def body(p_ref, d_ref, w2_ref, b2_ref, o_ref):
        ssum = p_ref[0] + p_ref[1]
        dsum = d_ref[0] + d_ref[1]
        norm = 1.0 / jnp.maximum(dsum, 1.0)
        agg = jnp.dot(ssum, w2_ref[...], preferred_element_type=jnp.float32)
        v = (agg + dsum * b2_ref[...]) * norm
        m = jnp.max(v, axis=1, keepdims=True)
        e = jnp.exp(v - m)
        lse = jnp.log(jnp.sum(e, axis=1, keepdims=True))
        o_ref[...] = (v - m) - lse

    return pl.pallas_call(
        body,
        grid=(n_pad // blk,),
        in_specs=[
            pl.BlockSpec((_NC, blk, h_dim), lambda i: (0, i, 0)),
            pl.BlockSpec((_NC, blk, 1), lambda i: (0, i, 0)),
            pl.BlockSpec((h_dim, c_dim), lambda i: (0, 0)),
            pl.BlockSpec((1, c_dim), lambda i: (0, 0)),
        ],
        out_specs=pl.BlockSpec((blk, c_dim), lambda i: (i, 0)),
        out_shape=jax.ShapeDtypeStruct((n_pad, c_dim), jnp.float32),
    )(s2, deg3, W2, b2.reshape(1, c_dim))


def kernel(x, adj, W1, b1, W2, b2):
    n, f_in = x.shape
    h_dim = W1.shape[1]
    c_dim = W2.shape[1]
    e = adj.shape[1]

    n_pad = ((n + 1023) // 1024) * 1024
    per_w = _NW * _K
    n_chunks = (e + per_w - 1) // per_w
    e_pad = n_chunks * per_w

    src = adj[0].astype(jnp.int32)
    dst = adj[1].astype(jnp.int32)
    srcp = jnp.concatenate(
        [src, jnp.zeros((e_pad - e,), jnp.int32)]).reshape(_NW, n_chunks, _K)
    dstp = jnp.concatenate(
        [dst, jnp.full((e_pad - e,), n, jnp.int32)]).reshape(_NW, n_chunks, _K)

    seg1 = _make_segsum(n, n_pad, f_in, n_chunks, with_deg=True)
    s1, deg = seg1(x, srcp, dstp)
    deg3 = deg.reshape(_NC, n_pad, 1)

    h = _tc_layer1(s1, deg3, W1, b1, blk=512)

    seg2 = _make_segsum(n_pad, n_pad, h_dim, n_chunks, with_deg=False)
    (s2,) = seg2(h, srcp, dstp)

    out = _tc_layer2(s2, deg3, W2, b2, blk=512)
    return out[:n]
